# baseline (speedup 1.0000x reference)
import functools

import jax
import jax.numpy as jnp
from jax import lax
from jax.experimental import pallas as pl
from jax.experimental.pallas import tpu as pltpu
from jax.experimental.pallas import tpu_sc as plsc

N = 10000
M = 10000
E = 320000
D = 128
OUT = 128

NC = 2
NS = 16
DH = D // NC
CH = 128
EPT = 20480
E_PAD = NS * EPT
NCHUNK = EPT // CH
MROWS = 10240
DUMMY = 10100
ACC_PT = MROWS // NS
OUT_PT = MROWS // NS
DCOL = 16
NBUF = 2


def _sc_segment_sum(rows2d, cols3d, nf16):
    mesh = plsc.VectorSubcoreMesh(
        core_axis_name="c", subcore_axis_name="s", num_cores=NC, num_subcores=NS
    )

    @functools.partial(
        pl.kernel,
        mesh=mesh,
        compiler_params=pltpu.CompilerParams(
            use_tc_tiling_on_sc=False, needs_layout_passes=False),
        out_type=(
            jax.ShapeDtypeStruct((NC, MROWS, DH), jnp.float32),
            jax.ShapeDtypeStruct((NC, MROWS, DCOL), jnp.float32),
        ),
        scratch_types=[
            pltpu.VMEM((2, NCHUNK, CH), jnp.int32),
            pltpu.VMEM((NBUF, CH, DH), jnp.bfloat16),
            pltpu.VMEM((NBUF, CH, DH), jnp.float32),
            pltpu.VMEM((CH, DCOL), jnp.float32),
            pltpu.VMEM_SHARED((MROWS, DH), jnp.float32),
            pltpu.VMEM_SHARED((MROWS, DCOL), jnp.float32),
            [pltpu.SemaphoreType.DMA] * NBUF,
            [pltpu.SemaphoreType.DMA] * NBUF,
        ],
    )
    def k(rows_hbm, cols_hbm, nf_hbm, s_out, d_out,
          idx_v, gbufv, ubufv, ones_v, acc_s, acc_d, semg, sems):
        cid = lax.axis_index("c")
        sid = lax.axis_index("s")
        gbufs = [gbufv.at[b] for b in range(NBUF)]
        ubufs = [ubufv.at[b] for b in range(NBUF)]

        zero16 = jnp.zeros((16,), jnp.float32)
        one16 = jnp.ones((16,), jnp.float32)

        def fill_z(i, carry):
            for j in range(DH // 16):
                ubufv[0, i, pl.ds(j * 16, 16)] = zero16
            ones_v[i, :] = zero16
            return carry

        lax.fori_loop(0, CH, fill_z, 0)

        base = sid * ACC_PT

        def zcopy(i, carry):
            pltpu.sync_copy(ubufs[0], acc_s.at[pl.ds(base + i * CH, CH), :])
            pltpu.sync_copy(ones_v, acc_d.at[pl.ds(base + i * CH, CH), :])
            return carry

        lax.fori_loop(0, ACC_PT // CH, zcopy, 0)

        def fill_ones(i, carry):
            ones_v[i, :] = one16
            return carry

        lax.fori_loop(0, CH, fill_ones, 0)
        plsc.subcore_barrier()

        pltpu.sync_copy(rows_hbm.at[pl.ds(sid * NCHUNK, NCHUNK), :], idx_v.at[0])
        pltpu.sync_copy(cols_hbm.at[cid, pl.ds(sid * NCHUNK, NCHUNK), :],
                        idx_v.at[1])

        for b in range(NBUF):
            pltpu.async_copy(nf_hbm.at[idx_v.at[1, b]], gbufs[b], semg[b])

        UNROLL = 8

        def unpack_chunk(b):
            def urow(r0, carry):
                for u in range(UNROLL):
                    r = r0 * UNROLL + u
                    for j in range(DH // 32):
                        x = gbufv[b, r, pl.ds(32 * j, 32)]
                        lo, hi = plsc.unpack(
                            x, format=plsc.PackFormat.INTERLEAVED,
                            preferred_element_type=jnp.float32)
                        ubufv[b, r, pl.ds(32 * j, 16)] = lo
                        ubufv[b, r, pl.ds(32 * j + 16, 16)] = hi
                return carry

            lax.fori_loop(0, CH // UNROLL, urow, 0)

        def stepn(i, carry):
            for b in range(NBUF):
                c = NBUF * i + b
                pltpu.make_async_copy(
                    nf_hbm.at[idx_v.at[1, c]], gbufs[b], semg[b]).wait()

                @pl.when(i > 0)
                def _():
                    pltpu.make_async_copy(
                        ubufs[b], acc_s.at[idx_v.at[0, c]], sems[b]).wait()

                    @pl.when(cid == (b % 2))
                    def _():
                        pltpu.make_async_copy(
                            ones_v, acc_d.at[idx_v.at[0, c]], sems[b]).wait()

                unpack_chunk(b)

                @pl.when(i < NCHUNK // NBUF - 1)
                def _():
                    pltpu.async_copy(
                        nf_hbm.at[idx_v.at[1, c + NBUF]], gbufs[b], semg[b])

                pltpu.async_copy(
                    ubufs[b], acc_s.at[idx_v.at[0, c]], sems[b], add=True)

                @pl.when(cid == (b % 2))
                def _():
                    pltpu.async_copy(
                        ones_v, acc_d.at[idx_v.at[0, c]], sems[b], add=True)

            return carry

        lax.fori_loop(0, NCHUNK // NBUF, stepn, 0)

        for b in range(NBUF):
            c = NCHUNK - NBUF + b
            pltpu.make_async_copy(
                ubufs[b], acc_s.at[idx_v.at[0, c]], sems[b]).wait()

            @pl.when(cid == (b % 2))
            def _():
                pltpu.make_async_copy(
                    ones_v, acc_d.at[idx_v.at[0, c]], sems[b]).wait()
        plsc.subcore_barrier()

        ob = sid * OUT_PT
        pltpu.sync_copy(acc_s.at[pl.ds(ob, OUT_PT), :],
                        s_out.at[cid, pl.ds(ob, OUT_PT), :])
        pltpu.sync_copy(acc_d.at[pl.ds(ob, OUT_PT), :],
                        d_out.at[cid, pl.ds(ob, OUT_PT), :])

    return k(rows2d, cols3d, nf16)


BM = 2000


def _tc_finish(s_part, d_part, W, b2):
    def body(s_ref, d_ref, dfull_ref, w_ref, b_ref, o_ref):
        degf = dfull_ref[0, 0:M, 0:1] + dfull_ref[1, 0:M, 0:1]
        delta = jnp.sum(jnp.log10(degf + 2.0)) / jnp.float32(M)
        s = jnp.concatenate([s_ref[0], s_ref[1]], axis=1)
        deg = d_ref[0, :, 0:1] + d_ref[1, :, 0:1]
        mean = s / jnp.where(deg > 0, deg, 1.0)
        logd = jnp.log10(deg + 2.0)
        scale = logd / delta
        dn = (((1,), (1,)), ((), ()))
        hp = dict(preferred_element_type=jnp.float32, precision=lax.Precision.HIGHEST)
        out = (lax.dot_general(mean, w_ref[:, 0:D], dn, **hp)
               + lax.dot_general(mean * scale, w_ref[:, D:2 * D], dn, **hp)
               + lax.dot_general(mean / scale, w_ref[:, 2 * D:3 * D], dn, **hp))
        out = out + b_ref[0:1, :]
        o_ref[...] = jnp.where(out > 0, out, 0.2 * out)

    return pl.pallas_call(
        body,
        grid=(M // BM,),
        in_specs=[
            pl.BlockSpec((NC, BM, DH), lambda i: (0, i, 0)),
            pl.BlockSpec((NC, BM, DCOL), lambda i: (0, i, 0)),
            pl.BlockSpec((NC, MROWS, DCOL), lambda i: (0, 0, 0)),
            pl.BlockSpec((OUT, 3 * D), lambda i: (0, 0)),
            pl.BlockSpec((1, OUT), lambda i: (0, 0)),
        ],
        out_specs=pl.BlockSpec((BM, OUT), lambda i: (i, 0)),
        out_shape=jax.ShapeDtypeStruct((M, OUT), jnp.float32),
    )(s_part, d_part, d_part, W, b2)


def kernel(edge_index, node_features, W, b):
    pad = E_PAD - E
    rows_p = jnp.concatenate(
        [edge_index[0], jnp.full((pad,), DUMMY, jnp.int32)]).reshape(E_PAD // CH, CH)
    cols_p = jnp.concatenate(
        [edge_index[1], jnp.zeros((pad,), jnp.int32)]).reshape(E_PAD // CH, CH)
    cols3d = jnp.stack([cols_p, cols_p + N])
    nf_half = (node_features.reshape(N, NC, DH)
               .swapaxes(0, 1).reshape(NC * N, DH))
    nf16 = (nf_half.astype(jnp.bfloat16)
            .reshape(NC * N * (DH // 32), 2, 16)
            .swapaxes(1, 2).reshape(NC * N, DH))
    s_part, d_part = _sc_segment_sum(rows_p, cols3d, nf16)
    return _tc_finish(s_part, d_part, W, b.reshape(1, OUT))


# speedup vs baseline: 1.9832x; 1.9832x over previous
import functools

import jax
import jax.numpy as jnp
from jax import lax
from jax.experimental import pallas as pl
from jax.experimental.pallas import tpu as pltpu
from jax.experimental.pallas import tpu_sc as plsc

N = 10000
M = 10000
E = 320000
D = 128
OUT = 128

NC = 2
NS = 16
DH = D // NC
CH = 128
EPT = 20480
E_PAD = NS * EPT
NCHUNK = EPT // CH
MROWS = 10240
DUMMY = 10100
ACC_PT = MROWS // NS
OUT_PT = MROWS // NS
DCOL = 16
NBUF = 4


def _sc_segment_sum(rows2d, cols3d, nf_half):
    mesh = plsc.VectorSubcoreMesh(
        core_axis_name="c", subcore_axis_name="s", num_cores=NC, num_subcores=NS
    )

    @functools.partial(
        pl.kernel,
        mesh=mesh,
        compiler_params=pltpu.CompilerParams(use_tc_tiling_on_sc=False),
        out_type=(
            jax.ShapeDtypeStruct((NC, MROWS, DH), jnp.float32),
            jax.ShapeDtypeStruct((NC, MROWS, DCOL), jnp.float32),
        ),
        scratch_types=[
            pltpu.VMEM((NCHUNK, CH), jnp.int32),
            pltpu.VMEM((NCHUNK, CH), jnp.int32),
            pltpu.VMEM((NBUF, CH, DH), jnp.float32),
            pltpu.VMEM((CH, DCOL), jnp.float32),
            pltpu.VMEM_SHARED((MROWS, DH), jnp.float32),
            pltpu.VMEM_SHARED((MROWS, DCOL), jnp.float32),
            [pltpu.SemaphoreType.DMA] * NBUF,
            [pltpu.SemaphoreType.DMA] * NBUF,
        ],
    )
    def k(rows_hbm, cols_hbm, nf_hbm, s_out, d_out,
          ridx_v, cidx_v, gbufv, ones_v, acc_s, acc_d,
          semg, sems):
        cid = lax.axis_index("c")
        sid = lax.axis_index("s")
        gbufs = [gbufv.at[b] for b in range(NBUF)]

        zero16 = jnp.zeros((16,), jnp.float32)
        one16 = jnp.ones((16,), jnp.float32)

        def fill_z(i, carry):
            for j in range(DH // 16):
                gbufv[0, i, pl.ds(j * 16, 16)] = zero16
            ones_v[i, :] = zero16
            return carry

        lax.fori_loop(0, CH, fill_z, 0)

        base = sid * ACC_PT

        def zcopy(i, carry):
            pltpu.sync_copy(gbufs[0], acc_s.at[pl.ds(base + i * CH, CH), :])
            pltpu.sync_copy(ones_v, acc_d.at[pl.ds(base + i * CH, CH), :])
            return carry

        lax.fori_loop(0, ACC_PT // CH, zcopy, 0)

        def fill_ones(i, carry):
            ones_v[i, :] = one16
            return carry

        lax.fori_loop(0, CH, fill_ones, 0)
        plsc.subcore_barrier()

        pltpu.sync_copy(rows_hbm.at[pl.ds(sid * NCHUNK, NCHUNK), :], ridx_v)
        pltpu.sync_copy(cols_hbm.at[cid, pl.ds(sid * NCHUNK, NCHUNK), :], cidx_v)

        for b in range(NBUF):
            pltpu.async_copy(nf_hbm.at[cidx_v.at[b]], gbufs[b], semg[b])

        def stepn(i, carry):
            for b in range(NBUF):
                c = NBUF * i + b
                pltpu.make_async_copy(
                    nf_hbm.at[cidx_v.at[c]], gbufs[b], semg[b]).wait()
                pltpu.async_copy(
                    gbufs[b], acc_s.at[ridx_v.at[c]], sems[b], add=True)

                @pl.when(cid == (b % 2))
                def _():
                    pltpu.async_copy(
                        ones_v, acc_d.at[ridx_v.at[c]], sems[b], add=True)

            @pl.when(i < NCHUNK // NBUF - 1)
            def _():
                for b in range(NBUF):
                    c = NBUF * i + b
                    pltpu.make_async_copy(
                        gbufs[b], acc_s.at[ridx_v.at[c]], sems[b]).wait()

                    @pl.when(cid == (b % 2))
                    def _():
                        pltpu.make_async_copy(
                            ones_v, acc_d.at[ridx_v.at[c]], sems[b]).wait()

                    pltpu.async_copy(
                        nf_hbm.at[cidx_v.at[c + NBUF]], gbufs[b], semg[b])

            return carry

        lax.fori_loop(0, NCHUNK // NBUF, stepn, 0)

        for b in range(NBUF):
            c = NCHUNK - NBUF + b
            pltpu.make_async_copy(
                gbufs[b], acc_s.at[ridx_v.at[c]], sems[b]).wait()

            @pl.when(cid == (b % 2))
            def _():
                pltpu.make_async_copy(
                    ones_v, acc_d.at[ridx_v.at[c]], sems[b]).wait()
        plsc.subcore_barrier()

        ob = sid * OUT_PT
        pltpu.sync_copy(acc_s.at[pl.ds(ob, OUT_PT), :],
                        s_out.at[cid, pl.ds(ob, OUT_PT), :])
        pltpu.sync_copy(acc_d.at[pl.ds(ob, OUT_PT), :],
                        d_out.at[cid, pl.ds(ob, OUT_PT), :])

    return k(rows2d, cols3d, nf_half)


BM = 2000


def _tc_finish(s_part, d_part, W, b2):
    def body(s_ref, d_ref, dfull_ref, w_ref, b_ref, o_ref):
        degf = dfull_ref[0, 0:M, 0:1] + dfull_ref[1, 0:M, 0:1]
        delta = jnp.sum(jnp.log10(degf + 2.0)) / jnp.float32(M)
        s = jnp.concatenate([s_ref[0], s_ref[1]], axis=1)
        deg = d_ref[0, :, 0:1] + d_ref[1, :, 0:1]
        mean = s / jnp.where(deg > 0, deg, 1.0)
        logd = jnp.log10(deg + 2.0)
        scale = logd / delta
        dn = (((1,), (1,)), ((), ()))
        hp = dict(preferred_element_type=jnp.float32, precision=lax.Precision.HIGHEST)
        out = (lax.dot_general(mean, w_ref[:, 0:D], dn, **hp)
               + lax.dot_general(mean * scale, w_ref[:, D:2 * D], dn, **hp)
               + lax.dot_general(mean / scale, w_ref[:, 2 * D:3 * D], dn, **hp))
        out = out + b_ref[0:1, :]
        o_ref[...] = jnp.where(out > 0, out, 0.2 * out)

    return pl.pallas_call(
        body,
        grid=(M // BM,),
        in_specs=[
            pl.BlockSpec((NC, BM, DH), lambda i: (0, i, 0)),
            pl.BlockSpec((NC, BM, DCOL), lambda i: (0, i, 0)),
            pl.BlockSpec((NC, MROWS, DCOL), lambda i: (0, 0, 0)),
            pl.BlockSpec((OUT, 3 * D), lambda i: (0, 0)),
            pl.BlockSpec((1, OUT), lambda i: (0, 0)),
        ],
        out_specs=pl.BlockSpec((BM, OUT), lambda i: (i, 0)),
        out_shape=jax.ShapeDtypeStruct((M, OUT), jnp.float32),
    )(s_part, d_part, d_part, W, b2)


def kernel(edge_index, node_features, W, b):
    pad = E_PAD - E
    rows_p = jnp.concatenate(
        [edge_index[0], jnp.full((pad,), DUMMY, jnp.int32)]).reshape(E_PAD // CH, CH)
    cols_p = jnp.concatenate(
        [edge_index[1], jnp.zeros((pad,), jnp.int32)]).reshape(E_PAD // CH, CH)
    cols3d = jnp.stack([cols_p, cols_p + N])
    nf_half = (node_features.reshape(N, NC, DH)
               .swapaxes(0, 1).reshape(NC * N, DH))
    s_part, d_part = _sc_segment_sum(rows_p, cols3d, nf_half)
    return _tc_finish(s_part, d_part, W, b.reshape(1, OUT))
